# baseline (speedup 1.0000x reference)
import functools

import jax
import jax.numpy as jnp
from jax import lax
from jax.experimental import pallas as pl
from jax.experimental.pallas import tpu as pltpu
from jax.experimental.pallas import tpu_sc as plsc

BATCH = 16384
EMBED_DIM = 64
NUM_CLASSES = 2
NUM_ROWS = 1000000

_info = plsc.get_sparse_core_info()
_NC, _NS, _L = _info.num_cores, _info.num_subcores, _info.num_lanes
_NW = _NC * _NS
_BPW = BATCH // _NW
_CHUNKS = _BPW // _L

_BW = 16384
_NBLK = (NUM_ROWS + _BW - 1) // _BW

_mesh = plsc.VectorSubcoreMesh(core_axis_name="c", subcore_axis_name="s")


def _dot_block(w_ref, tt_ref, o_ref):
    wd = w_ref[:, 0:1] - w_ref[:, 1:2]
    o_ref[...] = jnp.sum(tt_ref[...] * wd, axis=0)


_table_dot = pl.pallas_call(
    _dot_block,
    grid=(_NBLK,),
    in_specs=[
        pl.BlockSpec((EMBED_DIM, NUM_CLASSES), lambda i: (0, 0)),
        pl.BlockSpec((EMBED_DIM, _BW), lambda i: (0, i)),
    ],
    out_specs=pl.BlockSpec((_BW,), lambda i: (i,)),
    out_shape=jax.ShapeDtypeStruct((NUM_ROWS,), jnp.float32),
)


@functools.partial(
    pl.kernel,
    mesh=_mesh,
    compiler_params=pltpu.CompilerParams(needs_layout_passes=False,
                                         use_tc_tiling_on_sc=False),
    out_type=jax.ShapeDtypeStruct((BATCH * NUM_CLASSES,), jnp.float32),
    scratch_types=[
        pltpu.VMEM((_BPW,), jnp.int32),
        pltpu.VMEM((_BPW,), jnp.float32),
        pltpu.VMEM((16,), jnp.float32),
        pltpu.VMEM((_BPW * NUM_CLASSES,), jnp.float32),
        pltpu.SemaphoreType.DMA,
    ],
)
def _gather_sigmoid(idx_hbm, dbase_hbm, b_hbm, out_hbm,
                    idx_v, d_v, b_v, out_v, sem):
    wid = lax.axis_index("s") * _NC + lax.axis_index("c")
    base = wid * _BPW

    pltpu.sync_copy(idx_hbm.at[pl.ds(base, _BPW)], idx_v)
    pltpu.sync_copy(b_hbm, b_v)
    pltpu.async_copy(dbase_hbm.at[idx_v], d_v, sem).wait()

    lane = lax.iota(jnp.int32, 16)
    bvec = b_v[...]
    db = bvec[0] - bvec[1]
    ones = jnp.zeros((16,), jnp.float32) + 1.0

    def chunk(k, carry):
        d = d_v[pl.ds(k * 16, 16)] + db
        e = jnp.exp(-d)
        p0 = ones / (ones + e)
        p1 = 1.0 - p0
        ob = k * 32 + lane * 2
        plsc.store_scatter(out_v, [ob], p0)
        plsc.store_scatter(out_v, [ob + 1], p1)
        return carry

    lax.fori_loop(0, _CHUNKS, chunk, 0)
    pltpu.sync_copy(out_v, out_hbm.at[pl.ds(base * NUM_CLASSES,
                                            _BPW * NUM_CLASSES)])


def kernel(inputs, table, W, b):
    idx = inputs.astype(jnp.int32)
    b16 = jnp.pad(b.astype(jnp.float32), (0, 16 - NUM_CLASSES))
    dbase = _table_dot(W, table.T)
    out = _gather_sigmoid(idx, dbase, b16)
    return out.reshape(BATCH, NUM_CLASSES)


# speedup vs baseline: 1.1282x; 1.1282x over previous
import functools

import jax
import jax.numpy as jnp
from jax import lax
from jax.experimental import pallas as pl
from jax.experimental.pallas import tpu as pltpu
from jax.experimental.pallas import tpu_sc as plsc

BATCH = 16384
EMBED_DIM = 64
NUM_CLASSES = 2
NUM_ROWS = 1000000

_info = plsc.get_sparse_core_info()
_NC, _NS, _L = _info.num_cores, _info.num_subcores, _info.num_lanes
_NW = _NC * _NS
_BPW = BATCH // _NW
_CHUNKS = _BPW // _L

_BW = 32768
_NBLK = (NUM_ROWS + _BW - 1) // _BW

_mesh = plsc.VectorSubcoreMesh(core_axis_name="c", subcore_axis_name="s")


def _dot_block(w_ref, tt_ref, o_ref):
    wd = w_ref[:, 0:1] - w_ref[:, 1:2]
    o_ref[...] = jnp.sum(tt_ref[...] * wd, axis=0)


_table_dot = pl.pallas_call(
    _dot_block,
    grid=(_NBLK,),
    in_specs=[
        pl.BlockSpec((EMBED_DIM, NUM_CLASSES), lambda i: (0, 0)),
        pl.BlockSpec((EMBED_DIM, _BW), lambda i: (0, i)),
    ],
    out_specs=pl.BlockSpec((_BW,), lambda i: (i,)),
    out_shape=jax.ShapeDtypeStruct((NUM_ROWS,), jnp.float32),
)


@functools.partial(
    pl.kernel,
    mesh=_mesh,
    compiler_params=pltpu.CompilerParams(needs_layout_passes=False,
                                         use_tc_tiling_on_sc=False),
    out_type=jax.ShapeDtypeStruct((BATCH * NUM_CLASSES,), jnp.float32),
    scratch_types=[
        pltpu.VMEM((_BPW,), jnp.int32),
        pltpu.VMEM((_BPW,), jnp.float32),
        pltpu.VMEM((16,), jnp.float32),
        pltpu.VMEM((_BPW * NUM_CLASSES,), jnp.float32),
        pltpu.SemaphoreType.DMA,
    ],
)
def _gather_sigmoid(idx_hbm, dbase_hbm, b_hbm, out_hbm,
                    idx_v, d_v, b_v, out_v, sem):
    wid = lax.axis_index("s") * _NC + lax.axis_index("c")
    base = wid * _BPW

    pltpu.sync_copy(idx_hbm.at[pl.ds(base, _BPW)], idx_v)
    pltpu.sync_copy(b_hbm, b_v)
    pltpu.async_copy(dbase_hbm.at[idx_v], d_v, sem).wait()

    lane = lax.iota(jnp.int32, 16)
    bvec = b_v[...]
    db = bvec[0] - bvec[1]
    ones = jnp.zeros((16,), jnp.float32) + 1.0

    def chunk(k, carry):
        d = d_v[pl.ds(k * 16, 16)] + db
        e = jnp.exp(-d)
        p0 = ones / (ones + e)
        p1 = 1.0 - p0
        ob = k * 32 + lane * 2
        plsc.store_scatter(out_v, [ob], p0)
        plsc.store_scatter(out_v, [ob + 1], p1)
        return carry

    lax.fori_loop(0, _CHUNKS, chunk, 0)
    pltpu.sync_copy(out_v, out_hbm.at[pl.ds(base * NUM_CLASSES,
                                            _BPW * NUM_CLASSES)])


def kernel(inputs, table, W, b):
    idx = inputs.astype(jnp.int32)
    b16 = jnp.pad(b.astype(jnp.float32), (0, 16 - NUM_CLASSES))
    dbase = _table_dot(W, table.T)
    out = _gather_sigmoid(idx, dbase, b16)
    return out.reshape(BATCH, NUM_CLASSES)


# speedup vs baseline: 1.3244x; 1.1739x over previous
import functools

import jax
import jax.numpy as jnp
from jax import lax
from jax.experimental import pallas as pl
from jax.experimental.pallas import tpu as pltpu
from jax.experimental.pallas import tpu_sc as plsc

BATCH = 16384
EMBED_DIM = 64
NUM_CLASSES = 2
NUM_ROWS = 1000000

_info = plsc.get_sparse_core_info()
_NC, _NS, _L = _info.num_cores, _info.num_subcores, _info.num_lanes
_NW = _NC * _NS
_BPW = BATCH // _NW
_CHUNKS = _BPW // _L

_BW = 32768
_NBLK = (NUM_ROWS + _BW - 1) // _BW

_mesh = plsc.VectorSubcoreMesh(core_axis_name="c", subcore_axis_name="s")


def _dot_block(w_ref, b_ref, tt_ref, o_ref):
    wd = w_ref[:, 0:1] - w_ref[:, 1:2]
    db = b_ref[0] - b_ref[1]
    o_ref[...] = jnp.sum(tt_ref[...] * wd, axis=0) + db


_table_dot = pl.pallas_call(
    _dot_block,
    grid=(_NBLK,),
    in_specs=[
        pl.BlockSpec((EMBED_DIM, NUM_CLASSES), lambda i: (0, 0)),
        pl.BlockSpec((NUM_CLASSES,), lambda i: (0,)),
        pl.BlockSpec((EMBED_DIM, _BW), lambda i: (0, i)),
    ],
    out_specs=pl.BlockSpec((_BW,), lambda i: (i,)),
    out_shape=jax.ShapeDtypeStruct((NUM_ROWS,), jnp.float32),
)


@functools.partial(
    pl.kernel,
    mesh=_mesh,
    compiler_params=pltpu.CompilerParams(needs_layout_passes=False,
                                         use_tc_tiling_on_sc=False),
    out_type=jax.ShapeDtypeStruct((BATCH * NUM_CLASSES,), jnp.float32),
    scratch_types=[
        pltpu.VMEM((_BPW,), jnp.int32),
        pltpu.VMEM((_BPW,), jnp.float32),
        pltpu.VMEM((_BPW * NUM_CLASSES,), jnp.float32),
        pltpu.SemaphoreType.DMA,
    ],
)
def _gather_sigmoid(idx_hbm, dbase_hbm, out_hbm, idx_v, d_v, out_v, sem):
    wid = lax.axis_index("s") * _NC + lax.axis_index("c")
    base = wid * _BPW

    pltpu.sync_copy(idx_hbm.at[pl.ds(base, _BPW)], idx_v)
    pltpu.async_copy(dbase_hbm.at[idx_v], d_v, sem).wait()

    lane = lax.iota(jnp.int32, 16)
    ones = jnp.zeros((16,), jnp.float32) + 1.0

    def chunk(k, carry):
        d = d_v[pl.ds(k * 16, 16)]
        e = jnp.exp(-d)
        p0 = ones / (ones + e)
        p1 = 1.0 - p0
        ob = (k // 8) * 256 + (k % 8) * 16 + lane
        plsc.store_scatter(out_v, [ob], p0)
        plsc.store_scatter(out_v, [ob + 128], p1)
        return carry

    lax.fori_loop(0, _CHUNKS, chunk, 0)
    pltpu.sync_copy(out_v, out_hbm.at[pl.ds(base * NUM_CLASSES,
                                            _BPW * NUM_CLASSES)])


def kernel(inputs, table, W, b):
    idx = inputs.astype(jnp.int32)
    dbase = _table_dot(W, b, table.T)
    out = _gather_sigmoid(idx, dbase)
    out3 = out.reshape(BATCH // 128, NUM_CLASSES, 128)
    return out3.transpose(0, 2, 1).reshape(BATCH, NUM_CLASSES)


# speedup vs baseline: 1.3355x; 1.0084x over previous
import functools

import jax
import jax.numpy as jnp
from jax import lax
from jax.experimental import pallas as pl
from jax.experimental.pallas import tpu as pltpu
from jax.experimental.pallas import tpu_sc as plsc

BATCH = 16384
EMBED_DIM = 64
NUM_CLASSES = 2
NUM_ROWS = 1000000

_info = plsc.get_sparse_core_info()
_NC, _NS, _L = _info.num_cores, _info.num_subcores, _info.num_lanes
_NW = _NC * _NS
_BPW = BATCH // _NW
_CHUNKS = _BPW // _L

_BW = 40960
_NBLK = (NUM_ROWS + _BW - 1) // _BW

_mesh = plsc.VectorSubcoreMesh(core_axis_name="c", subcore_axis_name="s")


def _dot_block(w_ref, b_ref, tt_ref, o_ref):
    wd = w_ref[:, 0:1] - w_ref[:, 1:2]
    db = b_ref[0] - b_ref[1]
    o_ref[...] = jnp.sum(tt_ref[...] * wd, axis=0) + db


_table_dot = pl.pallas_call(
    _dot_block,
    grid=(_NBLK,),
    in_specs=[
        pl.BlockSpec((EMBED_DIM, NUM_CLASSES), lambda i: (0, 0)),
        pl.BlockSpec((NUM_CLASSES,), lambda i: (0,)),
        pl.BlockSpec((EMBED_DIM, _BW), lambda i: (0, i)),
    ],
    out_specs=pl.BlockSpec((_BW,), lambda i: (i,)),
    out_shape=jax.ShapeDtypeStruct((NUM_ROWS,), jnp.float32),
)


@functools.partial(
    pl.kernel,
    mesh=_mesh,
    compiler_params=pltpu.CompilerParams(needs_layout_passes=False,
                                         use_tc_tiling_on_sc=False),
    out_type=jax.ShapeDtypeStruct((BATCH * NUM_CLASSES,), jnp.float32),
    scratch_types=[
        pltpu.VMEM((_BPW,), jnp.int32),
        pltpu.VMEM((_BPW,), jnp.float32),
        pltpu.VMEM((_BPW * NUM_CLASSES,), jnp.float32),
        pltpu.SemaphoreType.DMA,
    ],
)
def _gather_sigmoid(idx_hbm, dbase_hbm, out_hbm, idx_v, d_v, out_v, sem):
    wid = lax.axis_index("s") * _NC + lax.axis_index("c")
    base = wid * _BPW

    pltpu.sync_copy(idx_hbm.at[pl.ds(base, _BPW)], idx_v)
    pltpu.async_copy(dbase_hbm.at[idx_v], d_v, sem).wait()

    lane = lax.iota(jnp.int32, 16)
    ones = jnp.zeros((16,), jnp.float32) + 1.0

    def chunk(k, carry):
        d = d_v[pl.ds(k * 16, 16)]
        e = jnp.exp(-d)
        p0 = ones / (ones + e)
        p1 = 1.0 - p0
        ob = (k // 8) * 256 + (k % 8) * 16 + lane
        plsc.store_scatter(out_v, [ob], p0)
        plsc.store_scatter(out_v, [ob + 128], p1)
        return carry

    lax.fori_loop(0, _CHUNKS, chunk, 0)
    pltpu.sync_copy(out_v, out_hbm.at[pl.ds(base * NUM_CLASSES,
                                            _BPW * NUM_CLASSES)])


def kernel(inputs, table, W, b):
    idx = inputs.astype(jnp.int32)
    dbase = _table_dot(W, b, table.T)
    out = _gather_sigmoid(idx, dbase)
    out3 = out.reshape(BATCH // 128, NUM_CLASSES, 128)
    return out3.transpose(0, 2, 1).reshape(BATCH, NUM_CLASSES)
